# baseline (speedup 1.0000x reference)
import functools
import math

import jax
import jax.numpy as jnp
from jax import lax
from jax.experimental import pallas as pl
from jax.experimental.pallas import tpu as pltpu
from jax.experimental.pallas import tpu_sc as plsc

N = 10000
E = 320000
D = 128
HD = 64
G = 64
NPAD = 10240
BLK = 2048
NGRID = NPAD // BLK

NC, NS = 2, 16
CH = 128
WCH = 128
EPT = E // NS
NBUF = 4
K = 158
NROWS = NPAD + 128
WR = NPAD // NS
WK = WR // WCH

_sc_mesh = plsc.VectorSubcoreMesh(
    core_axis_name="c", subcore_axis_name="s", num_cores=NC, num_subcores=NS)


@functools.partial(
    pl.kernel,
    out_type=jax.ShapeDtypeStruct((2 * NPAD, HD), jnp.float32),
    mesh=_sc_mesh,
    compiler_params=pltpu.CompilerParams(use_tc_tiling_on_sc=False),
    scratch_types=[
        pltpu.VMEM((K + 2, CH), jnp.int32),
        pltpu.VMEM((K, CH), jnp.int32),
        pltpu.VMEM((WK, WCH), jnp.int32),
        pltpu.VMEM((NBUF, CH, HD), jnp.float32),
        pltpu.VMEM((WCH, HD), jnp.float32),
        pltpu.VMEM_SHARED((NROWS, HD), jnp.float32),
        pltpu.SemaphoreType.DMA,
        pltpu.SemaphoreType.DMA,
        pltpu.SemaphoreType.DMA,
        pltpu.SemaphoreType.DMA,
        pltpu.SemaphoreType.DMA,
        pltpu.SemaphoreType.DMA,
        pltpu.SemaphoreType.DMA,
        pltpu.SemaphoreType.DMA,
        pltpu.SemaphoreType.DMA,
    ],
)
def _seg_sum_sc(h2_hbm, src2_hbm, dst_hbm, widx_hbm, zeros_hbm, out_hbm,
                srcv, dstv, widxv, rows, wbuf, acc, sem0, sem1, sem2, sem3,
                ssem0, ssem1, ssem2, ssem3, wsem):
    c = lax.axis_index("c")
    s = lax.axis_index("s")
    zr = NROWS // NS
    pltpu.sync_copy(zeros_hbm.at[pl.ds(s * zr, zr)], acc.at[pl.ds(s * zr, zr)])
    pltpu.sync_copy(src2_hbm.at[c].at[s], srcv.at[pl.ds(0, K)])
    pltpu.sync_copy(src2_hbm.at[c].at[s].at[pl.ds(0, 2)], srcv.at[pl.ds(K, 2)])
    pltpu.sync_copy(dst_hbm.at[s], dstv)
    pltpu.sync_copy(widx_hbm.at[c].at[s], widxv)
    plsc.subcore_barrier()

    def _g(j, b, sem):
        return pltpu.make_async_copy(h2_hbm.at[srcv.at[j]], rows.at[b], sem)

    def _s(j, b, sem):
        return pltpu.make_async_copy(rows.at[b], acc.at[dstv.at[j]], sem)

    _g(0, 0, sem0).start()
    _g(1, 1, sem1).start()
    _g(2, 2, sem2).start()
    _g(3, 3, sem3).start()

    def body(q, carry):
        base = 4 * q
        g0 = _g(base, 0, sem0)
        g1 = _g(base + 1, 1, sem1)
        g0.wait()
        s0 = _s(base, 0, ssem0)
        s0.start(add=True)
        g1.wait()
        s1 = _s(base + 1, 1, ssem1)
        s1.start(add=True)
        g2 = _g(base + 2, 2, sem2)
        g3 = _g(base + 3, 3, sem3)
        g2.wait()
        s2 = _s(base + 2, 2, ssem2)
        s2.start(add=True)
        g3.wait()
        s3 = _s(base + 3, 3, ssem3)
        s3.start(add=True)
        s0.wait()
        s1.wait()
        _g(base + 4, 0, sem0).start()
        _g(base + 5, 1, sem1).start()
        s2.wait()
        s3.wait()
        _g(base + 6, 2, sem2).start()
        _g(base + 7, 3, sem3).start()
        return carry

    lax.fori_loop(0, K // 4, body, 0)

    for t in range(K - (K // 4) * 4):
        j = (K // 4) * 4 + t
        _g(j, t, (sem0, sem1)[t]).wait()
        se = _s(j, t, ssem0)
        se.start(add=True)
        se.wait()
    _g(K, 2, sem2).wait()
    _g(K + 1, 3, sem3).wait()
    plsc.subcore_barrier()

    def wbody(k, carry):
        pltpu.sync_copy(acc.at[pl.ds(s * WR + k * WCH, WCH)], wbuf)
        cp = pltpu.make_async_copy(wbuf, out_hbm.at[widxv.at[k]], wsem)
        cp.start()
        cp.wait()
        return carry

    lax.fori_loop(0, WK, wbody, 0)


def _enc_body(xb, wt, b, ob):
    ob[...] = jnp.dot(xb[...], wt[...],
                      preferred_element_type=jnp.float32) + b[...]


_enc = pl.pallas_call(
    _enc_body,
    grid=(NGRID,),
    in_specs=[
        pl.BlockSpec((BLK, D), lambda i: (i, 0)),
        pl.BlockSpec((D, D), lambda i: (0, 0)),
        pl.BlockSpec((1, D), lambda i: (0, 0)),
    ],
    out_specs=pl.BlockSpec((BLK, D), lambda i: (i, 0)),
    out_shape=jax.ShapeDtypeStruct((NPAD, D), jnp.float32),
)


def _mlp_body(relu_out, hb, ab, w1t, b1, w2t, b2, ob):
    u = hb[...] + ab[...]
    t = jnp.dot(u, w1t[...], preferred_element_type=jnp.float32) + b1[...]
    t = jnp.maximum(t, 0.0)
    v = jnp.dot(t, w2t[...], preferred_element_type=jnp.float32) + b2[...]
    if relu_out:
        v = jnp.maximum(v, 0.0)
    ob[...] = v


def _make_mlp(relu_out):
    return pl.pallas_call(
        functools.partial(_mlp_body, relu_out),
        grid=(NGRID,),
        in_specs=[
            pl.BlockSpec((BLK, D), lambda i: (i, 0)),
            pl.BlockSpec((BLK, D), lambda i: (i, 0)),
            pl.BlockSpec((D, D), lambda i: (0, 0)),
            pl.BlockSpec((1, D), lambda i: (0, 0)),
            pl.BlockSpec((D, D), lambda i: (0, 0)),
            pl.BlockSpec((1, D), lambda i: (0, 0)),
        ],
        out_specs=pl.BlockSpec((BLK, D), lambda i: (i, 0)),
        out_shape=jax.ShapeDtypeStruct((NPAD, D), jnp.float32),
    )


_mlp_relu = _make_mlp(True)


def _pool_body(hb, ab, w1t, b1, w2t, b2, bb, lwt, lb, cwt, cb, ob, sums, cnt):
    i = pl.program_id(0)

    @pl.when(i == 0)
    def _():
        sums[...] = jnp.zeros_like(sums)
        cnt[...] = jnp.zeros_like(cnt)

    u = hb[...] + ab[...]
    t = jnp.dot(u, w1t[...], preferred_element_type=jnp.float32) + b1[...]
    t = jnp.maximum(t, 0.0)
    h2b = jnp.dot(t, w2t[...], preferred_element_type=jnp.float32) + b2[...]

    b = bb[0, 0, :]
    onehot = (lax.broadcasted_iota(jnp.int32, (G, BLK), 0)
              == b[None, :]).astype(jnp.float32)
    sums[...] += jnp.dot(onehot, h2b, preferred_element_type=jnp.float32)
    cnt[...] += jnp.sum(onehot, axis=1, keepdims=True)

    @pl.when(i == NGRID - 1)
    def _():
        xg = sums[...] / jnp.maximum(cnt[...], 1.0)
        y = jnp.dot(xg, lwt[...], preferred_element_type=jnp.float32) + lb[...]
        y = 0.5 * y * (1.0 + lax.erf(y * (1.0 / math.sqrt(2.0))))
        ob[...] = jnp.dot(y, cwt[...],
                          preferred_element_type=jnp.float32) + cb[...]


_pool = pl.pallas_call(
    _pool_body,
    grid=(NGRID,),
    in_specs=[
        pl.BlockSpec((BLK, D), lambda i: (i, 0)),
        pl.BlockSpec((BLK, D), lambda i: (i, 0)),
        pl.BlockSpec((D, D), lambda i: (0, 0)),
        pl.BlockSpec((1, D), lambda i: (0, 0)),
        pl.BlockSpec((D, D), lambda i: (0, 0)),
        pl.BlockSpec((1, D), lambda i: (0, 0)),
        pl.BlockSpec((1, 1, BLK), lambda i: (i, 0, 0)),
        pl.BlockSpec((D, D), lambda i: (0, 0)),
        pl.BlockSpec((1, D), lambda i: (0, 0)),
        pl.BlockSpec((D, D), lambda i: (0, 0)),
        pl.BlockSpec((1, D), lambda i: (0, 0)),
    ],
    out_specs=pl.BlockSpec((G, D), lambda i: (0, 0)),
    out_shape=jax.ShapeDtypeStruct((G, D), jnp.float32),
    scratch_shapes=[
        pltpu.VMEM((G, D), jnp.float32),
        pltpu.VMEM((G, 1), jnp.float32),
    ],
)


def _fold_bn(W, b, g, be):
    s = g * jnp.float32(1.0 / math.sqrt(1.0 + 1e-5))
    return W * s[:, None], b * s + be


def _seg_sum(h, src2, dst_p, widx, zeros):
    out = _seg_sum_sc(h.reshape(2 * NPAD, HD), src2, dst_p, widx, zeros)
    return out.reshape(NPAD, D)


def kernel(x, edge_index, batch, enc_W, enc_b,
           c0_W1, c0_b1, c0_g1, c0_be1, c0_W2, c0_b2, c0_g2, c0_be2,
           c1_W1, c1_b1, c1_g1, c1_be1, c1_W2, c1_b2, c1_g2, c1_be2,
           lin_W, lin_b, clf_W, clf_b):
    f32 = jnp.float32
    w10, b10 = _fold_bn(c0_W1, c0_b1, c0_g1, c0_be1)
    w20, b20 = _fold_bn(c0_W2, c0_b2, c0_g2, c0_be2)
    w11, b11 = _fold_bn(c1_W1, c1_b1, c1_g1, c1_be1)
    w21, b21 = _fold_bn(c1_W2, c1_b2, c1_g2, c1_be2)
    enc_Wt = enc_W.T
    w10t, w20t, w11t, w21t = w10.T, w20.T, w11.T, w21.T
    lin_Wt, clf_Wt = lin_W.T, clf_W.T
    row = lambda v: v.reshape(1, D).astype(f32)

    x_p = jnp.pad(x.astype(f32), ((0, NPAD - N), (0, 0)))
    src = edge_index[0].astype(jnp.int32).reshape(NS, EPT)
    dst = edge_index[1].astype(jnp.int32).reshape(NS, EPT)
    pad = K * CH - EPT
    srcr = jnp.pad(src, ((0, 0), (0, pad)))
    src2 = jnp.stack([2 * srcr, 2 * srcr + 1]).reshape(2, NS, K, CH)
    dst_p = jnp.pad(dst, ((0, 0), (0, pad)),
                    constant_values=NPAD).reshape(NS, K, CH)
    iw = jnp.arange(NPAD, dtype=jnp.int32).reshape(1, NS, WK, WCH)
    widx = jnp.concatenate([2 * iw, 2 * iw + 1], axis=0)
    batch3 = jnp.pad(batch.astype(jnp.int32), (0, NPAD - N),
                     constant_values=G).reshape(NGRID, 1, BLK)
    zeros = jnp.zeros((NROWS, HD), f32)

    h = _enc(x_p, enc_Wt, row(enc_b))
    a0 = _seg_sum(h, src2, dst_p, widx, zeros)
    h1 = _mlp_relu(h, a0, w10t, row(b10), w20t, row(b20))
    a1 = _seg_sum(h1, src2, dst_p, widx, zeros)
    out = _pool(h1, a1, w11t, row(b11), w21t, row(b21), batch3,
                lin_Wt, row(lin_b), clf_Wt, row(clf_b))
    return out


# speedup vs baseline: 1.0237x; 1.0237x over previous
import functools
import math

import jax
import jax.numpy as jnp
from jax import lax
from jax.experimental import pallas as pl
from jax.experimental.pallas import tpu as pltpu
from jax.experimental.pallas import tpu_sc as plsc

N = 10000
E = 320000
D = 128
HD = 64
G = 64
NPAD = 10240
BLK = 2048
NGRID = NPAD // BLK

NC, NS = 2, 16
CH = 128
WCH = 128
EPT = E // NS
NBUF = 4
K = 158
NROWS = NPAD + 128
WR = NPAD // NS
WK = WR // WCH

_sc_mesh = plsc.VectorSubcoreMesh(
    core_axis_name="c", subcore_axis_name="s", num_cores=NC, num_subcores=NS)


@functools.partial(
    pl.kernel,
    out_type=jax.ShapeDtypeStruct((2 * NPAD, HD), jnp.float32),
    mesh=_sc_mesh,
    compiler_params=pltpu.CompilerParams(use_tc_tiling_on_sc=False),
    scratch_types=[
        pltpu.VMEM((K + 2, CH), jnp.int32),
        pltpu.VMEM((K, CH), jnp.int32),
        pltpu.VMEM((WK, WCH), jnp.int32),
        pltpu.VMEM((NBUF, CH, HD), jnp.float32),
        pltpu.VMEM((WCH, HD), jnp.float32),
        pltpu.VMEM_SHARED((NROWS, HD), jnp.float32),
        pltpu.SemaphoreType.DMA,
        pltpu.SemaphoreType.DMA,
        pltpu.SemaphoreType.DMA,
        pltpu.SemaphoreType.DMA,
        pltpu.SemaphoreType.DMA,
        pltpu.SemaphoreType.DMA,
        pltpu.SemaphoreType.DMA,
    ],
)
def _seg_sum_sc(h2_hbm, src2_hbm, dst_hbm, widx_hbm, zeros_hbm, out_hbm,
                srcv, dstv, widxv, rows, wbuf, acc, sem0, sem1, sem2, sem3,
                ssem0, ssem1, wsem):
    c = lax.axis_index("c")
    s = lax.axis_index("s")
    zr = NROWS // NS
    pltpu.sync_copy(zeros_hbm.at[pl.ds(s * zr, zr)], acc.at[pl.ds(s * zr, zr)])
    pltpu.sync_copy(src2_hbm.at[c].at[s], srcv.at[pl.ds(0, K)])
    pltpu.sync_copy(src2_hbm.at[c].at[s].at[pl.ds(0, 2)], srcv.at[pl.ds(K, 2)])
    pltpu.sync_copy(dst_hbm.at[s], dstv)
    pltpu.sync_copy(widx_hbm.at[c].at[s], widxv)
    plsc.subcore_barrier()

    def _g(j, b, sem):
        return pltpu.make_async_copy(h2_hbm.at[srcv.at[j]], rows.at[b], sem)

    def _s(j, b, sem):
        return pltpu.make_async_copy(rows.at[b], acc.at[dstv.at[j]], sem)

    _g(0, 0, sem0).start()
    _g(1, 1, sem1).start()
    _g(2, 2, sem2).start()
    _g(3, 3, sem3).start()

    def body(q, carry):
        base = 4 * q
        g0 = _g(base, 0, sem0)
        g1 = _g(base + 1, 1, sem1)
        g0.wait()
        s0 = _s(base, 0, ssem0)
        s0.start(add=True)
        g1.wait()
        s1 = _s(base + 1, 1, ssem1)
        s1.start(add=True)
        s0.wait()
        s1.wait()
        _g(base + 4, 0, sem0).start()
        _g(base + 5, 1, sem1).start()
        g2 = _g(base + 2, 2, sem2)
        g3 = _g(base + 3, 3, sem3)
        g2.wait()
        s2 = _s(base + 2, 2, ssem0)
        s2.start(add=True)
        g3.wait()
        s3 = _s(base + 3, 3, ssem1)
        s3.start(add=True)
        s2.wait()
        s3.wait()
        _g(base + 6, 2, sem2).start()
        _g(base + 7, 3, sem3).start()
        return carry

    lax.fori_loop(0, K // 4, body, 0)

    for t in range(K - (K // 4) * 4):
        j = (K // 4) * 4 + t
        _g(j, t, (sem0, sem1)[t]).wait()
        se = _s(j, t, ssem0)
        se.start(add=True)
        se.wait()
    _g(K, 2, sem2).wait()
    _g(K + 1, 3, sem3).wait()
    plsc.subcore_barrier()

    def wbody(k, carry):
        pltpu.sync_copy(acc.at[pl.ds(s * WR + k * WCH, WCH)], wbuf)
        cp = pltpu.make_async_copy(wbuf, out_hbm.at[widxv.at[k]], wsem)
        cp.start()
        cp.wait()
        return carry

    lax.fori_loop(0, WK, wbody, 0)


def _enc_body(xb, wt, b, ob):
    ob[...] = jnp.dot(xb[...], wt[...],
                      preferred_element_type=jnp.float32) + b[...]


_enc = pl.pallas_call(
    _enc_body,
    grid=(NGRID,),
    in_specs=[
        pl.BlockSpec((BLK, D), lambda i: (i, 0)),
        pl.BlockSpec((D, D), lambda i: (0, 0)),
        pl.BlockSpec((1, D), lambda i: (0, 0)),
    ],
    out_specs=pl.BlockSpec((BLK, D), lambda i: (i, 0)),
    out_shape=jax.ShapeDtypeStruct((NPAD, D), jnp.float32),
)


def _mlp_body(relu_out, hb, ab, w1t, b1, w2t, b2, ob):
    u = hb[...] + ab[...]
    t = jnp.dot(u, w1t[...], preferred_element_type=jnp.float32) + b1[...]
    t = jnp.maximum(t, 0.0)
    v = jnp.dot(t, w2t[...], preferred_element_type=jnp.float32) + b2[...]
    if relu_out:
        v = jnp.maximum(v, 0.0)
    ob[...] = v


def _make_mlp(relu_out):
    return pl.pallas_call(
        functools.partial(_mlp_body, relu_out),
        grid=(NGRID,),
        in_specs=[
            pl.BlockSpec((BLK, D), lambda i: (i, 0)),
            pl.BlockSpec((BLK, D), lambda i: (i, 0)),
            pl.BlockSpec((D, D), lambda i: (0, 0)),
            pl.BlockSpec((1, D), lambda i: (0, 0)),
            pl.BlockSpec((D, D), lambda i: (0, 0)),
            pl.BlockSpec((1, D), lambda i: (0, 0)),
        ],
        out_specs=pl.BlockSpec((BLK, D), lambda i: (i, 0)),
        out_shape=jax.ShapeDtypeStruct((NPAD, D), jnp.float32),
    )


_mlp_relu = _make_mlp(True)


def _pool_body(hb, ab, w1t, b1, w2t, b2, bb, lwt, lb, cwt, cb, ob, sums, cnt):
    i = pl.program_id(0)

    @pl.when(i == 0)
    def _():
        sums[...] = jnp.zeros_like(sums)
        cnt[...] = jnp.zeros_like(cnt)

    u = hb[...] + ab[...]
    t = jnp.dot(u, w1t[...], preferred_element_type=jnp.float32) + b1[...]
    t = jnp.maximum(t, 0.0)
    h2b = jnp.dot(t, w2t[...], preferred_element_type=jnp.float32) + b2[...]

    b = bb[0, 0, :]
    onehot = (lax.broadcasted_iota(jnp.int32, (G, BLK), 0)
              == b[None, :]).astype(jnp.float32)
    sums[...] += jnp.dot(onehot, h2b, preferred_element_type=jnp.float32)
    cnt[...] += jnp.sum(onehot, axis=1, keepdims=True)

    @pl.when(i == NGRID - 1)
    def _():
        xg = sums[...] / jnp.maximum(cnt[...], 1.0)
        y = jnp.dot(xg, lwt[...], preferred_element_type=jnp.float32) + lb[...]
        y = 0.5 * y * (1.0 + lax.erf(y * (1.0 / math.sqrt(2.0))))
        ob[...] = jnp.dot(y, cwt[...],
                          preferred_element_type=jnp.float32) + cb[...]


_pool = pl.pallas_call(
    _pool_body,
    grid=(NGRID,),
    in_specs=[
        pl.BlockSpec((BLK, D), lambda i: (i, 0)),
        pl.BlockSpec((BLK, D), lambda i: (i, 0)),
        pl.BlockSpec((D, D), lambda i: (0, 0)),
        pl.BlockSpec((1, D), lambda i: (0, 0)),
        pl.BlockSpec((D, D), lambda i: (0, 0)),
        pl.BlockSpec((1, D), lambda i: (0, 0)),
        pl.BlockSpec((1, 1, BLK), lambda i: (i, 0, 0)),
        pl.BlockSpec((D, D), lambda i: (0, 0)),
        pl.BlockSpec((1, D), lambda i: (0, 0)),
        pl.BlockSpec((D, D), lambda i: (0, 0)),
        pl.BlockSpec((1, D), lambda i: (0, 0)),
    ],
    out_specs=pl.BlockSpec((G, D), lambda i: (0, 0)),
    out_shape=jax.ShapeDtypeStruct((G, D), jnp.float32),
    scratch_shapes=[
        pltpu.VMEM((G, D), jnp.float32),
        pltpu.VMEM((G, 1), jnp.float32),
    ],
)


def _fold_bn(W, b, g, be):
    s = g * jnp.float32(1.0 / math.sqrt(1.0 + 1e-5))
    return W * s[:, None], b * s + be


def _seg_sum(h, src2, dst_p, widx, zeros):
    out = _seg_sum_sc(h.reshape(2 * NPAD, HD), src2, dst_p, widx, zeros)
    return out.reshape(NPAD, D)


def kernel(x, edge_index, batch, enc_W, enc_b,
           c0_W1, c0_b1, c0_g1, c0_be1, c0_W2, c0_b2, c0_g2, c0_be2,
           c1_W1, c1_b1, c1_g1, c1_be1, c1_W2, c1_b2, c1_g2, c1_be2,
           lin_W, lin_b, clf_W, clf_b):
    f32 = jnp.float32
    w10, b10 = _fold_bn(c0_W1, c0_b1, c0_g1, c0_be1)
    w20, b20 = _fold_bn(c0_W2, c0_b2, c0_g2, c0_be2)
    w11, b11 = _fold_bn(c1_W1, c1_b1, c1_g1, c1_be1)
    w21, b21 = _fold_bn(c1_W2, c1_b2, c1_g2, c1_be2)
    enc_Wt = enc_W.T
    w10t, w20t, w11t, w21t = w10.T, w20.T, w11.T, w21.T
    lin_Wt, clf_Wt = lin_W.T, clf_W.T
    row = lambda v: v.reshape(1, D).astype(f32)

    x_p = jnp.pad(x.astype(f32), ((0, NPAD - N), (0, 0)))
    src = edge_index[0].astype(jnp.int32).reshape(NS, EPT)
    dst = edge_index[1].astype(jnp.int32).reshape(NS, EPT)
    pad = K * CH - EPT
    srcr = jnp.pad(src, ((0, 0), (0, pad)))
    src2 = jnp.stack([2 * srcr, 2 * srcr + 1]).reshape(2, NS, K, CH)
    dst_p = jnp.pad(dst, ((0, 0), (0, pad)),
                    constant_values=NPAD).reshape(NS, K, CH)
    iw = jnp.arange(NPAD, dtype=jnp.int32).reshape(1, NS, WK, WCH)
    widx = jnp.concatenate([2 * iw, 2 * iw + 1], axis=0)
    batch3 = jnp.pad(batch.astype(jnp.int32), (0, NPAD - N),
                     constant_values=G).reshape(NGRID, 1, BLK)
    zeros = jnp.zeros((NROWS, HD), f32)

    h = _enc(x_p, enc_Wt, row(enc_b))
    a0 = _seg_sum(h, src2, dst_p, widx, zeros)
    h1 = _mlp_relu(h, a0, w10t, row(b10), w20t, row(b20))
    a1 = _seg_sum(h1, src2, dst_p, widx, zeros)
    out = _pool(h1, a1, w11t, row(b11), w21t, row(b21), batch3,
                lin_Wt, row(lin_b), clf_Wt, row(clf_b))
    return out
